# baseline (speedup 1.0000x reference)
import functools

import jax
import jax.numpy as jnp
from jax import lax
from jax.experimental import pallas as pl
from jax.experimental.pallas import tpu as pltpu
from jax.experimental.pallas import tpu_sc as plsc

B, NL, L, D, V = 16, 2, 2048, 1024, 320
LANES = 16
CHUNKS = L // LANES
LB = 512
NBLK = L // LB


def _sc_weights_body(vq_hbm, len_hbm, w_hbm, vq_v, len_v, counts_v, prob_v):
    c = lax.axis_index("c")
    s = lax.axis_index("s")
    wid = s * 2 + c

    @pl.when(wid < B)
    def _():
        b = wid
        pltpu.sync_copy(vq_hbm.at[b], vq_v)
        pltpu.sync_copy(len_hbm, len_v)
        lens = len_v[...]
        length = lens.at[jnp.full((LANES,), b, jnp.int32)].get(
            mode="promise_in_bounds")

        iota = lax.iota(jnp.int32, LANES)
        ones_f = jnp.ones((LANES,), jnp.float32)
        zeros_f = jnp.zeros((LANES,), jnp.float32)

        def zbody(i, carry):
            counts_v[pl.ds(i * LANES, LANES)] = zeros_f
            return carry

        lax.fori_loop(0, (2 * V) // LANES, zbody, 0)

        def hbody(i, carry):
            rows = i * LANES + iota
            vx = plsc.load_gather(vq_v, [rows * 2])
            vy = plsc.load_gather(vq_v, [rows * 2 + 1])
            plsc.addupdate_scatter(counts_v, [vx], ones_f)
            plsc.addupdate_scatter(counts_v, [vy + V], ones_f)
            return carry

        lax.fori_loop(0, CHUNKS, hbody, 0)

        def pbody(i, acc):
            rows = i * LANES + iota
            vx = plsc.load_gather(vq_v, [rows * 2])
            vy = plsc.load_gather(vq_v, [rows * 2 + 1])
            fx = plsc.load_gather(counts_v, [vx])
            fy = plsc.load_gather(counts_v, [vy + V])
            mask = jnp.where(rows < length, 1.0, 0.0)
            p = mask / (fx + fy)
            prob_v[pl.ds(i * LANES, LANES)] = p
            return acc + p

        lax.fori_loop(0, CHUNKS, pbody, zeros_f)
        pltpu.sync_copy(prob_v, w_hbm.at[b])


@functools.cache
def _sc_weights_kernel():
    return pl.kernel(
        _sc_weights_body,
        out_type=jax.ShapeDtypeStruct((B, L), jnp.float32),
        mesh=plsc.VectorSubcoreMesh(core_axis_name="c", subcore_axis_name="s"),
        scratch_types=[
            pltpu.VMEM((2 * L,), jnp.int32),
            pltpu.VMEM((LANES,), jnp.int32),
            pltpu.VMEM((2 * V,), jnp.float32),
            pltpu.VMEM((L,), jnp.float32),
        ],
        compiler_params=pltpu.CompilerParams(
            needs_layout_passes=False, skip_device_barrier=True),
    )


def _tc_pool_body(nblk_ref, feat_ref, w_ref, out_ref, acc_ref):
    b = pl.program_id(0)
    l = pl.program_id(1)

    @pl.when(l == 0)
    def _():
        out_ref[...] = jnp.zeros_like(out_ref)
        acc_ref[0] = 0.0

    @pl.when(l < nblk_ref[b])
    def _():
        lmin = jnp.minimum(l, nblk_ref[b] - 1)
        f = feat_ref[0, 0]
        w = w_ref[b, lmin]
        out_ref[...] += jax.lax.dot(
            w.astype(jnp.bfloat16)[None, :], f.astype(jnp.bfloat16),
            preferred_element_type=jnp.float32)[None]
        acc_ref[0] += jnp.sum(w)

    @pl.when(l == NBLK - 1)
    def _():
        out_ref[...] = out_ref[...] / acc_ref[0]


def _tc_pool(nblk, input_feature, w):
    grid_spec = pltpu.PrefetchScalarGridSpec(
        num_scalar_prefetch=1,
        grid=(B, NBLK),
        in_specs=[
            pl.BlockSpec(
                (1, 1, LB, D),
                lambda b, l, nblk: (b, NL - 1, jnp.minimum(l, nblk[b] - 1), 0)),
            pl.BlockSpec(
                (B, NBLK, LB),
                lambda b, l, nblk: (0, 0, 0)),
        ],
        out_specs=pl.BlockSpec((1, 1, D), lambda b, l, nblk: (b, 0, 0)),
        scratch_shapes=[pltpu.SMEM((1,), jnp.float32)],
    )
    out = pl.pallas_call(
        _tc_pool_body,
        grid_spec=grid_spec,
        out_shape=jax.ShapeDtypeStruct((B, 1, D), jnp.float32),
        compiler_params=pltpu.CompilerParams(
            dimension_semantics=("parallel", "arbitrary")),
    )(nblk, input_feature, w)
    return out.reshape(B, D)


def kernel(input_feature, input_lengths, vq_indices):
    w = _sc_weights_kernel()(vq_indices.reshape(B, 2 * L), input_lengths)
    nblk = (input_lengths + LB - 1) // LB
    return _tc_pool(nblk, input_feature, w.reshape(B, NBLK, LB))


# speedup vs baseline: 1.0218x; 1.0218x over previous
import functools

import jax
import jax.numpy as jnp
from jax import lax
from jax.experimental import pallas as pl
from jax.experimental.pallas import tpu as pltpu
from jax.experimental.pallas import tpu_sc as plsc

B, NL, L, D, V = 16, 2, 2048, 1024, 320
LANES = 16
CHUNKS = L // LANES
LB = 512
NBLK = L // LB


def _sc_weights_body(vq_hbm, len_hbm, w_hbm, vq_v, len_v, counts_v, prob_v):
    c = lax.axis_index("c")
    s = lax.axis_index("s")
    wid = s * 2 + c

    @pl.when(wid < B)
    def _():
        b = wid
        pltpu.sync_copy(vq_hbm.at[b], vq_v)
        pltpu.sync_copy(len_hbm, len_v)
        lens = len_v[...]
        length = lens.at[jnp.full((LANES,), b, jnp.int32)].get(
            mode="promise_in_bounds")

        iota = lax.iota(jnp.int32, LANES)
        ones_f = jnp.ones((LANES,), jnp.float32)
        zeros_f = jnp.zeros((LANES,), jnp.float32)

        def zbody(i, carry):
            counts_v[pl.ds(i * LANES, LANES)] = zeros_f
            return carry

        lax.fori_loop(0, (2 * V) // LANES, zbody, 0)

        def hbody(i, carry):
            rows = i * LANES + iota
            vx = plsc.load_gather(vq_v, [rows * 2])
            vy = plsc.load_gather(vq_v, [rows * 2 + 1])
            plsc.addupdate_scatter(counts_v, [vx], ones_f)
            plsc.addupdate_scatter(counts_v, [vy + V], ones_f)
            return carry

        lax.fori_loop(0, CHUNKS, hbody, 0)

        def pbody(i, acc):
            rows = i * LANES + iota
            vx = plsc.load_gather(vq_v, [rows * 2])
            vy = plsc.load_gather(vq_v, [rows * 2 + 1])
            fx = plsc.load_gather(counts_v, [vx])
            fy = plsc.load_gather(counts_v, [vy + V])
            mask = jnp.where(rows < length, 1.0, 0.0)
            p = mask / (fx + fy)
            prob_v[pl.ds(i * LANES, LANES)] = p
            return acc + p

        lax.fori_loop(0, CHUNKS, pbody, zeros_f)
        pltpu.sync_copy(prob_v, w_hbm.at[b])


@functools.cache
def _sc_weights_kernel():
    return pl.kernel(
        _sc_weights_body,
        out_type=jax.ShapeDtypeStruct((B, L), jnp.float32),
        mesh=plsc.VectorSubcoreMesh(core_axis_name="c", subcore_axis_name="s"),
        scratch_types=[
            pltpu.VMEM((2 * L,), jnp.int32),
            pltpu.VMEM((LANES,), jnp.int32),
            pltpu.VMEM((2 * V,), jnp.float32),
            pltpu.VMEM((L,), jnp.float32),
        ],
        compiler_params=pltpu.CompilerParams(
            needs_layout_passes=False, skip_device_barrier=True),
    )


def _nblk(lens, b):
    return (lens[b] + LB - 1) // LB


def _tc_pool_body(lens_ref, feat_ref, w_ref, out_ref, acc_ref):
    b = pl.program_id(0)
    l = pl.program_id(1)
    nblk_b = _nblk(lens_ref, b)

    @pl.when(jnp.logical_and(b == 0, l == 0))
    def _():
        out_ref[...] = jnp.zeros_like(out_ref)

    @pl.when(l == 0)
    def _():
        acc_ref[0] = 0.0

    @pl.when(l < nblk_b)
    def _():
        lmin = jnp.minimum(l, nblk_b - 1)
        f = feat_ref[0, 0]
        w = w_ref[b, lmin]
        out_ref[0, pl.ds(b, 1), :] += jax.lax.dot(
            w.astype(jnp.bfloat16)[None, :], f.astype(jnp.bfloat16),
            preferred_element_type=jnp.float32)
        acc_ref[0] += jnp.sum(w)

    @pl.when(l == NBLK - 1)
    def _():
        out_ref[0, pl.ds(b, 1), :] = out_ref[0, pl.ds(b, 1), :] / acc_ref[0]


def _tc_pool(input_lengths, input_feature, w):
    grid_spec = pltpu.PrefetchScalarGridSpec(
        num_scalar_prefetch=1,
        grid=(B, NBLK),
        in_specs=[
            pl.BlockSpec(
                (1, 1, LB, D),
                lambda b, l, lens: (
                    b, NL - 1, jnp.minimum(l, _nblk(lens, b) - 1), 0)),
            pl.BlockSpec(
                (B, NBLK, LB),
                lambda b, l, lens: (0, 0, 0)),
        ],
        out_specs=pl.BlockSpec((1, B, D), lambda b, l, lens: (0, 0, 0)),
        scratch_shapes=[pltpu.SMEM((1,), jnp.float32)],
    )
    out = pl.pallas_call(
        _tc_pool_body,
        grid_spec=grid_spec,
        out_shape=jax.ShapeDtypeStruct((1, B, D), jnp.float32),
        compiler_params=pltpu.CompilerParams(
            dimension_semantics=("arbitrary", "arbitrary")),
    )(input_lengths, input_feature, w)
    return out.reshape(B, D)


def kernel(input_feature, input_lengths, vq_indices):
    w = _sc_weights_kernel()(vq_indices.reshape(B, 2 * L), input_lengths)
    return _tc_pool(input_lengths, input_feature, w.reshape(B, NBLK, LB))


# speedup vs baseline: 1.0621x; 1.0395x over previous
import functools

import jax
import jax.numpy as jnp
from jax import lax
from jax.experimental import pallas as pl
from jax.experimental.pallas import tpu as pltpu
from jax.experimental.pallas import tpu_sc as plsc

B, NL, L, D, V = 16, 2, 2048, 1024, 320
LANES = 16
CHUNKS = L // LANES
LB = 1024
NBLK = L // LB


def _sc_weights_body(vq_hbm, len_hbm, w_hbm, vq_v, len_v, counts_v, prob_v):
    c = lax.axis_index("c")
    s = lax.axis_index("s")
    wid = s * 2 + c

    @pl.when(wid < B)
    def _():
        b = wid
        pltpu.sync_copy(vq_hbm.at[b], vq_v)
        pltpu.sync_copy(len_hbm, len_v)
        lens = len_v[...]
        length = lens.at[jnp.full((LANES,), b, jnp.int32)].get(
            mode="promise_in_bounds")

        iota = lax.iota(jnp.int32, LANES)
        ones_f = jnp.ones((LANES,), jnp.float32)
        zeros_f = jnp.zeros((LANES,), jnp.float32)

        def zbody(i, carry):
            counts_v[pl.ds(i * LANES, LANES)] = zeros_f
            return carry

        lax.fori_loop(0, (2 * V) // LANES, zbody, 0)

        def hbody(i, carry):
            rows = i * LANES + iota
            vx = plsc.load_gather(vq_v, [rows * 2])
            vy = plsc.load_gather(vq_v, [rows * 2 + 1])
            plsc.addupdate_scatter(counts_v, [vx], ones_f)
            plsc.addupdate_scatter(counts_v, [vy + V], ones_f)
            return carry

        lax.fori_loop(0, CHUNKS, hbody, 0)

        def pbody(i, acc):
            rows = i * LANES + iota
            vx = plsc.load_gather(vq_v, [rows * 2])
            vy = plsc.load_gather(vq_v, [rows * 2 + 1])
            fx = plsc.load_gather(counts_v, [vx])
            fy = plsc.load_gather(counts_v, [vy + V])
            mask = jnp.where(rows < length, 1.0, 0.0)
            p = mask / (fx + fy)
            prob_v[pl.ds(i * LANES, LANES)] = p
            return acc + p

        lax.fori_loop(0, CHUNKS, pbody, zeros_f)
        pltpu.sync_copy(prob_v, w_hbm.at[b])


@functools.cache
def _sc_weights_kernel():
    return pl.kernel(
        _sc_weights_body,
        out_type=jax.ShapeDtypeStruct((B, L), jnp.float32),
        mesh=plsc.VectorSubcoreMesh(core_axis_name="c", subcore_axis_name="s"),
        scratch_types=[
            pltpu.VMEM((2 * L,), jnp.int32),
            pltpu.VMEM((LANES,), jnp.int32),
            pltpu.VMEM((2 * V,), jnp.float32),
            pltpu.VMEM((L,), jnp.float32),
        ],
        compiler_params=pltpu.CompilerParams(
            needs_layout_passes=False, skip_device_barrier=True),
    )


def _nblk(lens, b):
    return (lens[b] + LB - 1) // LB


def _tc_pool_body(lens_ref, feat_ref, w_ref, out_ref, acc_ref):
    b = pl.program_id(0)
    l = pl.program_id(1)
    nblk_b = _nblk(lens_ref, b)

    @pl.when(jnp.logical_and(b == 0, l == 0))
    def _():
        out_ref[...] = jnp.zeros_like(out_ref)

    @pl.when(l == 0)
    def _():
        acc_ref[0] = 0.0

    @pl.when(l < nblk_b)
    def _():
        lmin = jnp.minimum(l, nblk_b - 1)
        f = feat_ref[0, 0]
        w = w_ref[b, lmin]
        out_ref[0, pl.ds(b, 1), :] += jax.lax.dot(
            w.astype(jnp.bfloat16)[None, :], f.astype(jnp.bfloat16),
            preferred_element_type=jnp.float32)
        acc_ref[0] += jnp.sum(w)

    @pl.when(l == NBLK - 1)
    def _():
        out_ref[0, pl.ds(b, 1), :] = out_ref[0, pl.ds(b, 1), :] / acc_ref[0]


def _tc_pool(input_lengths, input_feature, w):
    grid_spec = pltpu.PrefetchScalarGridSpec(
        num_scalar_prefetch=1,
        grid=(B, NBLK),
        in_specs=[
            pl.BlockSpec(
                (1, 1, LB, D),
                lambda b, l, lens: (
                    b, NL - 1, jnp.minimum(l, _nblk(lens, b) - 1), 0)),
            pl.BlockSpec(
                (B, NBLK, LB),
                lambda b, l, lens: (0, 0, 0)),
        ],
        out_specs=pl.BlockSpec((1, B, D), lambda b, l, lens: (0, 0, 0)),
        scratch_shapes=[pltpu.SMEM((1,), jnp.float32)],
    )
    out = pl.pallas_call(
        _tc_pool_body,
        grid_spec=grid_spec,
        out_shape=jax.ShapeDtypeStruct((1, B, D), jnp.float32),
        compiler_params=pltpu.CompilerParams(
            dimension_semantics=("arbitrary", "arbitrary")),
    )(input_lengths, input_feature, w)
    return out.reshape(B, D)


def kernel(input_feature, input_lengths, vq_indices):
    w = _sc_weights_kernel()(vq_indices.reshape(B, 2 * L), input_lengths)
    return _tc_pool(input_lengths, input_feature, w.reshape(B, NBLK, LB))


# speedup vs baseline: 1.0776x; 1.0146x over previous
import functools

import jax
import jax.numpy as jnp
from jax import lax
from jax.experimental import pallas as pl
from jax.experimental.pallas import tpu as pltpu
from jax.experimental.pallas import tpu_sc as plsc

B, NL, L, D, V = 16, 2, 2048, 1024, 320
LANES = 16
CHUNKS = L // LANES
LB = 1024
NBLK = L // LB


HALF = L // 2
HCHUNKS = HALF // LANES


def _sc_weights_body(vq_hbm, len_hbm, w_hbm, vq_v, len_v, counts_v, prob_v,
                     partner_v, shared_v):
    c = lax.axis_index("c")
    s = lax.axis_index("s")
    j = s // 2
    h = s % 2
    b = c * 8 + j

    pltpu.sync_copy(vq_hbm.at[b, pl.ds(h * L, L)], vq_v)
    pltpu.sync_copy(len_hbm, len_v)
    lens = len_v[...]
    length = lens.at[jnp.full((LANES,), b, jnp.int32)].get(
        mode="promise_in_bounds")

    iota = lax.iota(jnp.int32, LANES)
    ones_f = jnp.ones((LANES,), jnp.float32)
    zeros_f = jnp.zeros((LANES,), jnp.float32)

    def zbody(i, carry):
        counts_v[pl.ds(i * LANES, LANES)] = zeros_f
        return carry

    lax.fori_loop(0, (2 * V) // LANES, zbody, 0)

    def hbody(i, carry):
        rows = i * LANES + iota
        vx = plsc.load_gather(vq_v, [rows * 2])
        vy = plsc.load_gather(vq_v, [rows * 2 + 1])
        plsc.addupdate_scatter(counts_v, [vx], ones_f)
        plsc.addupdate_scatter(counts_v, [vy + V], ones_f)
        return carry

    lax.fori_loop(0, HCHUNKS, hbody, 0)

    pltpu.sync_copy(counts_v, shared_v.at[s])
    plsc.subcore_barrier()
    pltpu.sync_copy(shared_v.at[s + 1 - 2 * h], partner_v)

    def mbody(i, carry):
        sl = pl.ds(i * LANES, LANES)
        counts_v[sl] = counts_v[sl] + partner_v[sl]
        return carry

    lax.fori_loop(0, (2 * V) // LANES, mbody, 0)

    base = h * HALF

    def pbody(i, acc):
        rows = i * LANES + iota
        vx = plsc.load_gather(vq_v, [rows * 2])
        vy = plsc.load_gather(vq_v, [rows * 2 + 1])
        fx = plsc.load_gather(counts_v, [vx])
        fy = plsc.load_gather(counts_v, [vy + V])
        mask = jnp.where(base + rows < length, 1.0, 0.0)
        p = mask / (fx + fy)
        prob_v[pl.ds(i * LANES, LANES)] = p
        return acc + p

    lax.fori_loop(0, HCHUNKS, pbody, zeros_f)
    pltpu.sync_copy(prob_v, w_hbm.at[b, pl.ds(base, HALF)])


@functools.cache
def _sc_weights_kernel():
    return pl.kernel(
        _sc_weights_body,
        out_type=jax.ShapeDtypeStruct((B, L), jnp.float32),
        mesh=plsc.VectorSubcoreMesh(core_axis_name="c", subcore_axis_name="s"),
        scratch_types=[
            pltpu.VMEM((L,), jnp.int32),
            pltpu.VMEM((LANES,), jnp.int32),
            pltpu.VMEM((2 * V,), jnp.float32),
            pltpu.VMEM((HALF,), jnp.float32),
            pltpu.VMEM((2 * V,), jnp.float32),
            pltpu.VMEM_SHARED((16, 2 * V), jnp.float32),
        ],
        compiler_params=pltpu.CompilerParams(
            needs_layout_passes=False, skip_device_barrier=True),
    )


def _nblk(lens, b):
    return (lens[b] + LB - 1) // LB


def _tc_pool_body(lens_ref, feat_ref, w_ref, out_ref, acc_ref):
    b = pl.program_id(0)
    l = pl.program_id(1)
    nblk_b = _nblk(lens_ref, b)

    @pl.when(jnp.logical_and(b == 0, l == 0))
    def _():
        out_ref[...] = jnp.zeros_like(out_ref)

    @pl.when(l == 0)
    def _():
        acc_ref[0] = 0.0

    @pl.when(l < nblk_b)
    def _():
        lmin = jnp.minimum(l, nblk_b - 1)
        f = feat_ref[0, 0]
        w = w_ref[b, lmin]
        out_ref[0, pl.ds(b, 1), :] += jax.lax.dot(
            w.astype(jnp.bfloat16)[None, :], f.astype(jnp.bfloat16),
            preferred_element_type=jnp.float32)
        acc_ref[0] += jnp.sum(w)

    @pl.when(l == NBLK - 1)
    def _():
        out_ref[0, pl.ds(b, 1), :] = out_ref[0, pl.ds(b, 1), :] / acc_ref[0]


def _tc_pool(input_lengths, input_feature, w):
    grid_spec = pltpu.PrefetchScalarGridSpec(
        num_scalar_prefetch=1,
        grid=(B, NBLK),
        in_specs=[
            pl.BlockSpec(
                (1, 1, LB, D),
                lambda b, l, lens: (
                    b, NL - 1, jnp.minimum(l, _nblk(lens, b) - 1), 0)),
            pl.BlockSpec(
                (B, NBLK, LB),
                lambda b, l, lens: (0, 0, 0)),
        ],
        out_specs=pl.BlockSpec((1, B, D), lambda b, l, lens: (0, 0, 0)),
        scratch_shapes=[pltpu.SMEM((1,), jnp.float32)],
    )
    out = pl.pallas_call(
        _tc_pool_body,
        grid_spec=grid_spec,
        out_shape=jax.ShapeDtypeStruct((1, B, D), jnp.float32),
        compiler_params=pltpu.CompilerParams(
            dimension_semantics=("arbitrary", "arbitrary")),
    )(input_lengths, input_feature, w)
    return out.reshape(B, D)


def kernel(input_feature, input_lengths, vq_indices):
    w = _sc_weights_kernel()(vq_indices.reshape(B, 2 * L), input_lengths)
    return _tc_pool(input_lengths, input_feature, w.reshape(B, NBLK, LB))


# speedup vs baseline: 1.1033x; 1.0238x over previous
import functools

import jax
import jax.numpy as jnp
from jax import lax
from jax.experimental import pallas as pl
from jax.experimental.pallas import tpu as pltpu
from jax.experimental.pallas import tpu_sc as plsc

B, NL, L, D, V = 16, 2, 2048, 1024, 320
LANES = 16
CHUNKS = L // LANES
LB = 1024
NBLK = L // LB


HALF = L // 2
HCHUNKS = HALF // LANES


def _sc_weights_body(vq_hbm, len_hbm, w_hbm, vq_v, len_v, counts_v, prob_v,
                     partner_v, shared_v):
    c = lax.axis_index("c")
    s = lax.axis_index("s")
    j = s // 2
    h = s % 2
    b = c * 8 + j

    pltpu.sync_copy(vq_hbm.at[b, pl.ds(h * L, L)], vq_v)
    pltpu.sync_copy(len_hbm, len_v)
    lens = len_v[...]
    length = lens.at[jnp.full((LANES,), b, jnp.int32)].get(
        mode="promise_in_bounds")

    iota = lax.iota(jnp.int32, LANES)
    ones_f = jnp.ones((LANES,), jnp.float32)
    zeros_f = jnp.zeros((LANES,), jnp.float32)

    def zbody(i, carry):
        counts_v[pl.ds(i * LANES, LANES)] = zeros_f
        return carry

    lax.fori_loop(0, (2 * V) // LANES, zbody, 0)

    def hbody(i, carry):
        rows = i * LANES + iota
        vx = plsc.load_gather(vq_v, [rows * 2])
        vy = plsc.load_gather(vq_v, [rows * 2 + 1])
        plsc.addupdate_scatter(counts_v, [vx], ones_f)
        plsc.addupdate_scatter(counts_v, [vy + V], ones_f)
        return carry

    lax.fori_loop(0, HCHUNKS, hbody, 0)

    pltpu.sync_copy(counts_v, shared_v.at[s])
    plsc.subcore_barrier()
    pltpu.sync_copy(shared_v.at[s + 1 - 2 * h], partner_v)

    def mbody(i, carry):
        sl = pl.ds(i * LANES, LANES)
        counts_v[sl] = counts_v[sl] + partner_v[sl]
        return carry

    lax.fori_loop(0, (2 * V) // LANES, mbody, 0)

    base = h * HALF

    def pbody(i, acc):
        rows = i * LANES + iota
        vx = plsc.load_gather(vq_v, [rows * 2])
        vy = plsc.load_gather(vq_v, [rows * 2 + 1])
        fx = plsc.load_gather(counts_v, [vx])
        fy = plsc.load_gather(counts_v, [vy + V])
        mask = jnp.where(base + rows < length, 1.0, 0.0)
        p = mask / (fx + fy)
        prob_v[pl.ds(i * LANES, LANES)] = p
        return acc + p

    lax.fori_loop(0, HCHUNKS, pbody, zeros_f)
    pltpu.sync_copy(prob_v, w_hbm.at[b, h])


@functools.cache
def _sc_weights_kernel():
    return pl.kernel(
        _sc_weights_body,
        out_type=jax.ShapeDtypeStruct((B, NBLK, LB), jnp.float32),
        mesh=plsc.VectorSubcoreMesh(core_axis_name="c", subcore_axis_name="s"),
        scratch_types=[
            pltpu.VMEM((L,), jnp.int32),
            pltpu.VMEM((LANES,), jnp.int32),
            pltpu.VMEM((2 * V,), jnp.float32),
            pltpu.VMEM((HALF,), jnp.float32),
            pltpu.VMEM((2 * V,), jnp.float32),
            pltpu.VMEM_SHARED((16, 2 * V), jnp.float32),
        ],
        compiler_params=pltpu.CompilerParams(needs_layout_passes=False),
    )


def _nblk(lens, b):
    return (lens[b] + LB - 1) // LB


def _tc_pool_body(lens_ref, feat_ref, w_ref, out_ref, acc_ref):
    b = pl.program_id(0)
    l = pl.program_id(1)
    nblk_b = _nblk(lens_ref, b)

    @pl.when(jnp.logical_and(b == 0, l == 0))
    def _():
        out_ref[...] = jnp.zeros_like(out_ref)

    @pl.when(l == 0)
    def _():
        acc_ref[0] = 0.0

    @pl.when(l < nblk_b)
    def _():
        lmin = jnp.minimum(l, nblk_b - 1)
        f = feat_ref[0, 0]
        w = w_ref[b, lmin]
        out_ref[0, pl.ds(b, 1), :] += jax.lax.dot(
            w.astype(jnp.bfloat16)[None, :], f.astype(jnp.bfloat16),
            preferred_element_type=jnp.float32)
        acc_ref[0] += jnp.sum(w)

    @pl.when(l == NBLK - 1)
    def _():
        out_ref[0, pl.ds(b, 1), :] = out_ref[0, pl.ds(b, 1), :] / acc_ref[0]


def _tc_pool(input_lengths, input_feature, w):
    grid_spec = pltpu.PrefetchScalarGridSpec(
        num_scalar_prefetch=1,
        grid=(B, NBLK),
        in_specs=[
            pl.BlockSpec(
                (1, 1, LB, D),
                lambda b, l, lens: (
                    b, NL - 1, jnp.minimum(l, _nblk(lens, b) - 1), 0)),
            pl.BlockSpec(
                (B, NBLK, LB),
                lambda b, l, lens: (0, 0, 0)),
        ],
        out_specs=pl.BlockSpec((1, B, D), lambda b, l, lens: (0, 0, 0)),
        scratch_shapes=[pltpu.SMEM((1,), jnp.float32)],
    )
    out = pl.pallas_call(
        _tc_pool_body,
        grid_spec=grid_spec,
        out_shape=jax.ShapeDtypeStruct((1, B, D), jnp.float32),
        compiler_params=pltpu.CompilerParams(
            dimension_semantics=("arbitrary", "arbitrary")),
    )(input_lengths, input_feature, w)
    return out.reshape(B, D)


def kernel(input_feature, input_lengths, vq_indices):
    w = _sc_weights_kernel()(vq_indices.reshape(B, 2 * L), input_lengths)
    return _tc_pool(input_lengths, input_feature, w)


# speedup vs baseline: 1.1286x; 1.0230x over previous
import functools

import jax
import jax.numpy as jnp
from jax import lax
from jax.experimental import pallas as pl
from jax.experimental.pallas import tpu as pltpu
from jax.experimental.pallas import tpu_sc as plsc

B, NL, L, D, V = 16, 2, 2048, 1024, 320
LANES = 16
CHUNKS = L // LANES
LB = 1024
NBLK = L // LB


HALF = L // 2
HCHUNKS = HALF // LANES


def _sc_weights_body(vq_hbm, w_hbm, vq_v, counts_v, prob_v,
                     partner_v, shared_v):
    c = lax.axis_index("c")
    s = lax.axis_index("s")
    j = s // 2
    h = s % 2
    b = c * 8 + j

    pltpu.sync_copy(vq_hbm.at[b, pl.ds(h * L, L)], vq_v)

    iota = lax.iota(jnp.int32, LANES)
    ones_f = jnp.ones((LANES,), jnp.float32)
    zeros_f = jnp.zeros((LANES,), jnp.float32)

    def zbody(i, carry):
        counts_v[pl.ds(i * LANES, LANES)] = zeros_f
        return carry

    lax.fori_loop(0, (2 * V) // LANES, zbody, 0)

    def hbody(i, carry):
        rows = i * LANES + iota
        vx = plsc.load_gather(vq_v, [rows * 2])
        vy = plsc.load_gather(vq_v, [rows * 2 + 1])
        plsc.addupdate_scatter(counts_v, [vx], ones_f)
        plsc.addupdate_scatter(counts_v, [vy + V], ones_f)
        return carry

    lax.fori_loop(0, HCHUNKS, hbody, 0)

    pltpu.sync_copy(counts_v, shared_v.at[s])
    plsc.subcore_barrier()
    pltpu.sync_copy(shared_v.at[s + 1 - 2 * h], partner_v)

    def mbody(i, carry):
        sl = pl.ds(i * LANES, LANES)
        counts_v[sl] = counts_v[sl] + partner_v[sl]
        return carry

    lax.fori_loop(0, (2 * V) // LANES, mbody, 0)

    def pbody(i, acc):
        rows = i * LANES + iota
        vx = plsc.load_gather(vq_v, [rows * 2])
        vy = plsc.load_gather(vq_v, [rows * 2 + 1])
        fx = plsc.load_gather(counts_v, [vx])
        fy = plsc.load_gather(counts_v, [vy + V])
        p = ones_f / (fx + fy)
        prob_v[pl.ds(i * LANES, LANES)] = p
        return acc + p

    lax.fori_loop(0, HCHUNKS, pbody, zeros_f)
    pltpu.sync_copy(prob_v, w_hbm.at[b, h])


@functools.cache
def _sc_weights_kernel():
    return pl.kernel(
        _sc_weights_body,
        out_type=jax.ShapeDtypeStruct((B, NBLK, LB), jnp.float32),
        mesh=plsc.VectorSubcoreMesh(core_axis_name="c", subcore_axis_name="s"),
        scratch_types=[
            pltpu.VMEM((L,), jnp.int32),
            pltpu.VMEM((2 * V,), jnp.float32),
            pltpu.VMEM((HALF,), jnp.float32),
            pltpu.VMEM((2 * V,), jnp.float32),
            pltpu.VMEM_SHARED((16, 2 * V), jnp.float32),
        ],
        compiler_params=pltpu.CompilerParams(needs_layout_passes=False),
    )


def _nblk(lens, b):
    return (lens[b] + LB - 1) // LB


def _tc_pool_body(lens_ref, feat_ref, w_ref, out_ref, acc_ref):
    b = pl.program_id(0)
    l = pl.program_id(1)
    nblk_b = _nblk(lens_ref, b)

    @pl.when(jnp.logical_and(b == 0, l == 0))
    def _():
        out_ref[...] = jnp.zeros_like(out_ref)

    @pl.when(l == 0)
    def _():
        acc_ref[0] = 0.0

    @pl.when(l < nblk_b)
    def _():
        lmin = jnp.minimum(l, nblk_b - 1)
        f = feat_ref[0, 0]
        w = w_ref[b, lmin][None, :]
        pos = lmin * LB + jax.lax.broadcasted_iota(jnp.int32, (1, LB), 1)
        w = jnp.where(pos < lens_ref[b], w, 0.0)
        out_ref[0, pl.ds(b, 1), :] += jax.lax.dot(
            w.astype(jnp.bfloat16), f.astype(jnp.bfloat16),
            preferred_element_type=jnp.float32)
        acc_ref[0] += jnp.sum(w)

    @pl.when(l == NBLK - 1)
    def _():
        out_ref[0, pl.ds(b, 1), :] = out_ref[0, pl.ds(b, 1), :] / acc_ref[0]


def _tc_pool(input_lengths, input_feature, w):
    grid_spec = pltpu.PrefetchScalarGridSpec(
        num_scalar_prefetch=1,
        grid=(B, NBLK),
        in_specs=[
            pl.BlockSpec(
                (1, 1, LB, D),
                lambda b, l, lens: (
                    b, NL - 1, jnp.minimum(l, _nblk(lens, b) - 1), 0)),
            pl.BlockSpec(
                (B, NBLK, LB),
                lambda b, l, lens: (0, 0, 0)),
        ],
        out_specs=pl.BlockSpec((1, B, D), lambda b, l, lens: (0, 0, 0)),
        scratch_shapes=[pltpu.SMEM((1,), jnp.float32)],
    )
    out = pl.pallas_call(
        _tc_pool_body,
        grid_spec=grid_spec,
        out_shape=jax.ShapeDtypeStruct((1, B, D), jnp.float32),
        compiler_params=pltpu.CompilerParams(
            dimension_semantics=("arbitrary", "arbitrary")),
    )(input_lengths, input_feature, w)
    return out.reshape(B, D)


def kernel(input_feature, input_lengths, vq_indices):
    w = _sc_weights_kernel()(vq_indices.reshape(B, 2 * L))
    return _tc_pool(input_lengths, input_feature, w)
